# baseline (device time: 53303 ns/iter reference)
import os

import jax
import jax.numpy as jnp
from jax import lax
from jax.experimental import pallas as pl
from jax.experimental.pallas import tpu as pltpu

ABLATE = int(os.environ.get("ABLATE", "0"))

N_DEV = 4
XCH = 4

_GELU_C = 0.7978845608028654


def _gelu(y):
    return 0.5 * y * (1.0 + jnp.tanh(_GELU_C * (y + 0.044715 * y * y * y)))


def kernel(x, w_mat):
    m_per, k = x.shape
    _, n = w_mat.shape
    n_per = n // N_DEV
    xrows = m_per // XCH

    def body(
        x_hbm,
        w_hbm,
        out_ref,
        x_bf,
        w_bf,
        xs,
        ws,
        x_sems,
        w_sems,
        send_buf,
        recv_buf,
        send_sems,
        recv_sems,
    ):
        me = lax.axis_index("i")

        def xdma(ci, slot):
            return pltpu.make_async_copy(
                x_hbm.at[pl.ds(ci * xrows, xrows), :],
                xs.at[slot],
                x_sems.at[slot],
            )

        def wdma(c, slot):
            return pltpu.make_async_copy(
                w_hbm.at[:, pl.ds(c * n_per, n_per)],
                ws.at[slot],
                w_sems.at[slot],
            )

        xdma(0, 0).start()
        wdma((me + 1) % N_DEV, 0).start()
        xdma(1, 1).start()

        barrier_sem = pltpu.get_barrier_semaphore()
        for d in range(1, N_DEV):
            pl.semaphore_signal(
                barrier_sem,
                inc=1,
                device_id=((me + d) % N_DEV,),
                device_id_type=pl.DeviceIdType.MESH,
            )
        pl.semaphore_wait(barrier_sem, N_DEV - 1)

        offs = [1, 3, 2, 0]
        for idx, off in enumerate(offs):
            c = (me + off) % N_DEV
            slot = idx % 2
            if idx + 1 < N_DEV:
                wdma((me + offs[idx + 1]) % N_DEV, 1 - slot).start()
            wdma(c, slot).wait()
            w_bf[slot] = ws[slot].astype(jnp.bfloat16)

            for r in range(XCH):
                if idx == 0:
                    xslot = r % 2
                    xdma(r, xslot).wait()
                    x_bf[pl.ds(r * xrows, xrows), :] = xs[xslot].astype(
                        jnp.bfloat16
                    )
                    if r + 2 < XCH:
                        xdma(r + 2, xslot).start()

                y = jnp.dot(
                    x_bf[pl.ds(r * xrows, xrows), :],
                    w_bf[slot],
                    preferred_element_type=jnp.float32,
                )
                y = _gelu(y)

                if off == 0:
                    out_ref[pl.ds(me * m_per + r * xrows, xrows), :] = y
                else:
                    send_buf[idx, r] = y.astype(jnp.bfloat16)
                    if ABLATE == 0:
                        pltpu.make_async_remote_copy(
                            src_ref=send_buf.at[idx, r],
                            dst_ref=recv_buf.at[me, r],
                            send_sem=send_sems.at[idx, r],
                            recv_sem=recv_sems.at[me, r],
                            device_id=(c,),
                            device_id_type=pl.DeviceIdType.MESH,
                        ).start()

        for off in [3, 2, 1]:
            s = (me + off) % N_DEV
            for r in range(XCH):
                if ABLATE == 0:
                    pltpu.make_async_remote_copy(
                        src_ref=send_buf.at[0, r],
                        dst_ref=recv_buf.at[s, r],
                        send_sem=send_sems.at[0, r],
                        recv_sem=recv_sems.at[s, r],
                        device_id=(s,),
                        device_id_type=pl.DeviceIdType.MESH,
                    ).wait_recv()
                out_ref[pl.ds(s * m_per + r * xrows, xrows), :] = recv_buf[
                    s, r
                ].astype(jnp.float32)

        if ABLATE == 0:
            for idx, off in enumerate(offs[:3]):
                c = (me + off) % N_DEV
                for r in range(XCH):
                    pltpu.make_async_remote_copy(
                        src_ref=send_buf.at[idx, r],
                        dst_ref=recv_buf.at[me, r],
                        send_sem=send_sems.at[idx, r],
                        recv_sem=recv_sems.at[me, r],
                        device_id=(c,),
                        device_id_type=pl.DeviceIdType.MESH,
                    ).wait_send()

    return pl.pallas_call(
        body,
        out_shape=jax.ShapeDtypeStruct((N_DEV * m_per, n_per), jnp.float32),
        in_specs=[
            pl.BlockSpec(memory_space=pltpu.MemorySpace.HBM),
            pl.BlockSpec(memory_space=pltpu.MemorySpace.HBM),
        ],
        out_specs=pl.BlockSpec(memory_space=pltpu.VMEM),
        scratch_shapes=[
            pltpu.VMEM((m_per, k), jnp.bfloat16),
            pltpu.VMEM((2, k, n_per), jnp.bfloat16),
            pltpu.VMEM((2, xrows, k), jnp.float32),
            pltpu.VMEM((2, k, n_per), jnp.float32),
            pltpu.SemaphoreType.DMA((2,)),
            pltpu.SemaphoreType.DMA((2,)),
            pltpu.VMEM((N_DEV, XCH, xrows, n_per), jnp.bfloat16),
            pltpu.VMEM((N_DEV, XCH, xrows, n_per), jnp.bfloat16),
            pltpu.SemaphoreType.DMA((N_DEV, XCH)),
            pltpu.SemaphoreType.DMA((N_DEV, XCH)),
        ],
        compiler_params=pltpu.CompilerParams(
            collective_id=0,
            vmem_limit_bytes=100 * 1024 * 1024,
        ),
    )(x, w_mat)


# device time: 37554 ns/iter; 1.4194x vs baseline; 1.4194x over previous
import os

import jax
import jax.numpy as jnp
from jax import lax
from jax.experimental import pallas as pl
from jax.experimental.pallas import tpu as pltpu

ABLATE = int(os.environ.get("ABLATE", "0"))
COMM_ON = ABLATE in (0, 5)

N_DEV = 4
XCH = 4

_GELU_C = 0.7978845608028654


def _gelu(y):
    return 0.5 * y * (1.0 + jnp.tanh(_GELU_C * (y + 0.044715 * y * y * y)))


def kernel(x, w_mat):
    m_per, k = x.shape
    _, n = w_mat.shape
    n_per = n // N_DEV
    xrows = m_per // XCH

    def body(
        x_hbm,
        w_hbm,
        out_ref,
        x_bf,
        w_bf,
        xs,
        ws,
        x_sems,
        w_sems,
        send_buf,
        recv_buf,
        send_sems,
        recv_sems,
    ):
        me = lax.axis_index("i")

        def xdma(ci, slot):
            return pltpu.make_async_copy(
                x_hbm.at[pl.ds(ci * xrows, xrows), :],
                xs.at[slot],
                x_sems.at[slot],
            )

        def wdma(c, slot):
            return pltpu.make_async_copy(
                w_hbm.at[:, pl.ds(c * n_per, n_per)],
                ws.at[slot],
                w_sems.at[slot],
            )

        if ABLATE != 5:
            xdma(0, 0).start()
            wdma((me + 1) % N_DEV, 0).start()
            xdma(1, 1).start()

        barrier_sem = pltpu.get_barrier_semaphore()
        for d in range(1, N_DEV):
            pl.semaphore_signal(
                barrier_sem,
                inc=1,
                device_id=((me + d) % N_DEV,),
                device_id_type=pl.DeviceIdType.MESH,
            )
        pl.semaphore_wait(barrier_sem, N_DEV - 1)

        offs = [1, 3, 2, 0]
        if ABLATE == 5:
            for idx, off in enumerate(offs[:3]):
                c = (me + off) % N_DEV
                for r in range(XCH):
                    pltpu.make_async_remote_copy(
                        src_ref=send_buf.at[idx, r],
                        dst_ref=recv_buf.at[me, r],
                        send_sem=send_sems.at[idx, r],
                        recv_sem=recv_sems.at[me, r],
                        device_id=(c,),
                        device_id_type=pl.DeviceIdType.MESH,
                    ).start()
        for idx, off in enumerate(offs if ABLATE != 5 else []):
            c = (me + off) % N_DEV
            slot = idx % 2
            if idx + 1 < N_DEV:
                wdma((me + offs[idx + 1]) % N_DEV, 1 - slot).start()
            wdma(c, slot).wait()
            w_bf[slot] = ws[slot].astype(jnp.bfloat16)

            for r in range(XCH):
                if idx == 0:
                    xslot = r % 2
                    xdma(r, xslot).wait()
                    x_bf[pl.ds(r * xrows, xrows), :] = xs[xslot].astype(
                        jnp.bfloat16
                    )
                    if r + 2 < XCH:
                        xdma(r + 2, xslot).start()

                y = jnp.dot(
                    x_bf[pl.ds(r * xrows, xrows), :],
                    w_bf[slot],
                    preferred_element_type=jnp.float32,
                )
                y = _gelu(y)

                if off == 0:
                    out_ref[pl.ds(me * m_per + r * xrows, xrows), :] = y
                else:
                    send_buf[idx, r] = y.astype(jnp.bfloat16)
                    if COMM_ON:
                        pltpu.make_async_remote_copy(
                            src_ref=send_buf.at[idx, r],
                            dst_ref=recv_buf.at[me, r],
                            send_sem=send_sems.at[idx, r],
                            recv_sem=recv_sems.at[me, r],
                            device_id=(c,),
                            device_id_type=pl.DeviceIdType.MESH,
                        ).start()

        for off in [3, 2, 1]:
            s = (me + off) % N_DEV
            for r in range(XCH):
                if COMM_ON:
                    pltpu.make_async_remote_copy(
                        src_ref=send_buf.at[0, r],
                        dst_ref=recv_buf.at[s, r],
                        send_sem=send_sems.at[0, r],
                        recv_sem=recv_sems.at[s, r],
                        device_id=(s,),
                        device_id_type=pl.DeviceIdType.MESH,
                    ).wait_recv()
                out_ref[pl.ds(s * m_per + r * xrows, xrows), :] = recv_buf[
                    s, r
                ].astype(jnp.float32)

        if COMM_ON:
            for idx, off in enumerate(offs[:3]):
                c = (me + off) % N_DEV
                for r in range(XCH):
                    pltpu.make_async_remote_copy(
                        src_ref=send_buf.at[idx, r],
                        dst_ref=recv_buf.at[me, r],
                        send_sem=send_sems.at[idx, r],
                        recv_sem=recv_sems.at[me, r],
                        device_id=(c,),
                        device_id_type=pl.DeviceIdType.MESH,
                    ).wait_send()

    return pl.pallas_call(
        body,
        out_shape=jax.ShapeDtypeStruct((N_DEV * m_per, n_per), jnp.float32),
        in_specs=[
            pl.BlockSpec(memory_space=pltpu.MemorySpace.HBM),
            pl.BlockSpec(memory_space=pltpu.MemorySpace.HBM),
        ],
        out_specs=pl.BlockSpec(memory_space=pltpu.VMEM),
        scratch_shapes=[
            pltpu.VMEM((m_per, k), jnp.bfloat16),
            pltpu.VMEM((2, k, n_per), jnp.bfloat16),
            pltpu.VMEM((2, xrows, k), jnp.float32),
            pltpu.VMEM((2, k, n_per), jnp.float32),
            pltpu.SemaphoreType.DMA((2,)),
            pltpu.SemaphoreType.DMA((2,)),
            pltpu.VMEM((N_DEV, XCH, xrows, n_per), jnp.bfloat16),
            pltpu.VMEM((N_DEV, XCH, xrows, n_per), jnp.bfloat16),
            pltpu.SemaphoreType.DMA((N_DEV, XCH)),
            pltpu.SemaphoreType.DMA((N_DEV, XCH)),
        ],
        compiler_params=pltpu.CompilerParams(
            collective_id=0,
            vmem_limit_bytes=100 * 1024 * 1024,
        ),
    )(x, w_mat)


# device time: 26141 ns/iter; 2.0391x vs baseline; 1.4366x over previous
import os

import jax
import jax.numpy as jnp
from jax import lax
from jax.experimental import pallas as pl
from jax.experimental.pallas import tpu as pltpu

ABLATE = int(os.environ.get("ABLATE", "0"))
COMM_ON = ABLATE in (0, 5, 6, 7)
LIVE_SEND = {6: [1, 3], 7: [1]}.get(ABLATE, [1, 3, 2])
LIVE_RECV = {6: [3, 1], 7: [3]}.get(ABLATE, [3, 2, 1])

N_DEV = 4
XCH = 4

_GELU_C = 0.7978845608028654


def _gelu(y):
    return 0.5 * y * (1.0 + jnp.tanh(_GELU_C * (y + 0.044715 * y * y * y)))


def kernel(x, w_mat):
    m_per, k = x.shape
    _, n = w_mat.shape
    n_per = n // N_DEV
    xrows = m_per // XCH

    def body(
        x_hbm,
        w_hbm,
        out_ref,
        x_bf,
        w_bf,
        xs,
        ws,
        x_sems,
        w_sems,
        send_buf,
        recv_buf,
        send_sems,
        recv_sems,
    ):
        me = lax.axis_index("i")

        def xdma(ci, slot):
            return pltpu.make_async_copy(
                x_hbm.at[pl.ds(ci * xrows, xrows), :],
                xs.at[slot],
                x_sems.at[slot],
            )

        def wdma(c, slot):
            return pltpu.make_async_copy(
                w_hbm.at[:, pl.ds(c * n_per, n_per)],
                ws.at[slot],
                w_sems.at[slot],
            )

        if ABLATE < 5:
            xdma(0, 0).start()
            wdma((me + 1) % N_DEV, 0).start()
            xdma(1, 1).start()

        barrier_sem = pltpu.get_barrier_semaphore()
        for d in range(1, N_DEV):
            pl.semaphore_signal(
                barrier_sem,
                inc=1,
                device_id=((me + d) % N_DEV,),
                device_id_type=pl.DeviceIdType.MESH,
            )
        pl.semaphore_wait(barrier_sem, N_DEV - 1)

        offs = [1, 3, 2, 0]
        if ABLATE >= 5:
            for idx, off in enumerate(LIVE_SEND):
                c = (me + off) % N_DEV
                for r in range(XCH):
                    pltpu.make_async_remote_copy(
                        src_ref=send_buf.at[idx, r],
                        dst_ref=recv_buf.at[me, r],
                        send_sem=send_sems.at[idx, r],
                        recv_sem=recv_sems.at[me, r],
                        device_id=(c,),
                        device_id_type=pl.DeviceIdType.MESH,
                    ).start()
        for idx, off in enumerate(offs if ABLATE < 5 else []):
            c = (me + off) % N_DEV
            slot = idx % 2
            if idx + 1 < N_DEV:
                wdma((me + offs[idx + 1]) % N_DEV, 1 - slot).start()
            wdma(c, slot).wait()
            w_bf[slot] = ws[slot].astype(jnp.bfloat16)

            for r in range(XCH):
                if idx == 0:
                    xslot = r % 2
                    xdma(r, xslot).wait()
                    x_bf[pl.ds(r * xrows, xrows), :] = xs[xslot].astype(
                        jnp.bfloat16
                    )
                    if r + 2 < XCH:
                        xdma(r + 2, xslot).start()

                y = jnp.dot(
                    x_bf[pl.ds(r * xrows, xrows), :],
                    w_bf[slot],
                    preferred_element_type=jnp.float32,
                )
                y = _gelu(y)

                if off == 0:
                    out_ref[pl.ds(me * m_per + r * xrows, xrows), :] = y
                else:
                    send_buf[idx, r] = y.astype(jnp.bfloat16)
                    if COMM_ON:
                        pltpu.make_async_remote_copy(
                            src_ref=send_buf.at[idx, r],
                            dst_ref=recv_buf.at[me, r],
                            send_sem=send_sems.at[idx, r],
                            recv_sem=recv_sems.at[me, r],
                            device_id=(c,),
                            device_id_type=pl.DeviceIdType.MESH,
                        ).start()

        for off in LIVE_RECV:
            s = (me + off) % N_DEV
            for r in range(XCH):
                if COMM_ON:
                    pltpu.make_async_remote_copy(
                        src_ref=send_buf.at[0, r],
                        dst_ref=recv_buf.at[s, r],
                        send_sem=send_sems.at[0, r],
                        recv_sem=recv_sems.at[s, r],
                        device_id=(s,),
                        device_id_type=pl.DeviceIdType.MESH,
                    ).wait_recv()
                out_ref[pl.ds(s * m_per + r * xrows, xrows), :] = recv_buf[
                    s, r
                ].astype(jnp.float32)

        if COMM_ON:
            for idx, off in enumerate(LIVE_SEND):
                c = (me + off) % N_DEV
                for r in range(XCH):
                    pltpu.make_async_remote_copy(
                        src_ref=send_buf.at[idx, r],
                        dst_ref=recv_buf.at[me, r],
                        send_sem=send_sems.at[idx, r],
                        recv_sem=recv_sems.at[me, r],
                        device_id=(c,),
                        device_id_type=pl.DeviceIdType.MESH,
                    ).wait_send()

    return pl.pallas_call(
        body,
        out_shape=jax.ShapeDtypeStruct((N_DEV * m_per, n_per), jnp.float32),
        in_specs=[
            pl.BlockSpec(memory_space=pltpu.MemorySpace.HBM),
            pl.BlockSpec(memory_space=pltpu.MemorySpace.HBM),
        ],
        out_specs=pl.BlockSpec(memory_space=pltpu.VMEM),
        scratch_shapes=[
            pltpu.VMEM((m_per, k), jnp.bfloat16),
            pltpu.VMEM((2, k, n_per), jnp.bfloat16),
            pltpu.VMEM((2, xrows, k), jnp.float32),
            pltpu.VMEM((2, k, n_per), jnp.float32),
            pltpu.SemaphoreType.DMA((2,)),
            pltpu.SemaphoreType.DMA((2,)),
            pltpu.VMEM((N_DEV, XCH, xrows, n_per), jnp.bfloat16),
            pltpu.VMEM((N_DEV, XCH, xrows, n_per), jnp.bfloat16),
            pltpu.SemaphoreType.DMA((N_DEV, XCH)),
            pltpu.SemaphoreType.DMA((N_DEV, XCH)),
        ],
        compiler_params=pltpu.CompilerParams(
            collective_id=0,
            vmem_limit_bytes=100 * 1024 * 1024,
        ),
    )(x, w_mat)
